# Initial kernel scaffold; baseline (speedup 1.0000x reference)
import jax
import jax.numpy as jnp
from jax.experimental import pallas as pl


def kernel(r_ij, edge_index, W0, b0, Wd, W1, b1, W2, b2, W3, b3, Wv, Wdd):
    raise NotImplementedError("write your pallas kernel here")


# speedup vs baseline: 33.5415x; 33.5415x over previous
import functools

import jax
import jax.numpy as jnp
from jax import lax
from jax.experimental import pallas as pl
from jax.experimental.pallas import tpu as pltpu
from jax.experimental.pallas import tpu_sc as plsc

N_NODES = 10000
E = 160000
R0 = 5.0

NC = 2
NS = 16
EB = 256
BLOCKS_PER_TILE = 40
E_PAD = NS * BLOCKS_PER_TILE * EB
ROWS_PER_TILE = 632
N_ACC = NS * ROWS_PER_TILE
PHI = 160
PHI_PAD = 256
BE = 2048
BN = 400


def _leaky(x):
    return jnp.where(x >= 0, x, 0.1 * x)


def _phi_body(r_ref, w0t, b0, wdt, w1t, b1, w2t, b2, w3t, b3, phi_ref):
    r = r_ref[...]
    d = jnp.sqrt(jnp.sum(r * r, axis=1, keepdims=True) + 1e-12)
    x = d / R0
    mu = lax.broadcasted_iota(jnp.int32, (1, 8), 1).astype(jnp.float32) / 7.0
    g = jnp.exp(-0.5 * ((x - mu) * 8.0) ** 2)
    h = jnp.dot(g, w0t[...], preferred_element_type=jnp.float32) + b0[...]
    direct = jnp.dot(h, wdt[...], preferred_element_type=jnp.float32)
    y = _leaky(jnp.dot(h, w1t[...], preferred_element_type=jnp.float32) + b1[...])
    y = _leaky(jnp.dot(y, w2t[...], preferred_element_type=jnp.float32) + b2[...])
    y = jnp.dot(y, w3t[...], preferred_element_type=jnp.float32) + b3[...]
    enc = direct + y
    rs = r * (7.0 / R0)
    n2 = jnp.sum(rs * rs, axis=1, keepdims=True)
    rr = rs / jnp.sqrt(1.0 + n2)
    xc, yc, zc = rr[:, 0:1], rr[:, 1:2], rr[:, 2:3]
    zero = jnp.zeros((enc.shape[0], PHI_PAD - PHI), jnp.float32)
    phi_ref[...] = jnp.concatenate(
        [enc, xc * enc, yc * enc, zc * enc,
         (xc * xc) * enc, (xc * yc) * enc, (xc * zc) * enc,
         (yc * yc) * enc, (yc * zc) * enc, (zc * zc) * enc, zero], axis=1)


def _stage1(r_pad, w0t, b0, wdt, w1t, b1, w2t, b2, w3t, b3):
    full = lambda s: pl.BlockSpec(s, lambda i: (0,) * len(s))
    return pl.pallas_call(
        _phi_body,
        grid=(E_PAD // BE,),
        in_specs=[
            pl.BlockSpec((BE, 3), lambda i: (i, 0)),
            full((8, 16)), full((1, 16)), full((16, 16)),
            full((16, 32)), full((1, 32)), full((32, 32)), full((1, 32)),
            full((32, 16)), full((1, 16)),
        ],
        out_specs=pl.BlockSpec((BE, PHI_PAD), lambda i: (i, 0)),
        out_shape=jax.ShapeDtypeStruct((E_PAD, PHI_PAD), jnp.float32),
    )(r_pad, w0t, b0, wdt, w1t, b1, w2t, b2, w3t, b3)


def _sc_scatter_body(phi_hbm, src_hbm, zero_hbm, out_hbm, idx_v, phi_v, acc):
    c = lax.axis_index("c")
    s = lax.axis_index("s")
    rbase = s * ROWS_PER_TILE
    col = c * 128
    pltpu.sync_copy(zero_hbm.at[pl.ds(rbase, ROWS_PER_TILE), :],
                    acc.at[pl.ds(rbase, ROWS_PER_TILE), :])
    plsc.subcore_barrier()

    def body(i, carry):
        base = s * (BLOCKS_PER_TILE * EB) + i * EB
        pltpu.sync_copy(
            src_hbm.at[pl.ds(s * (BLOCKS_PER_TILE * EB // 128) + i * (EB // 128),
                             EB // 128), :],
            idx_v)
        pltpu.sync_copy(phi_hbm.at[pl.ds(base, EB), pl.ds(col, 128)], phi_v)
        for j in range(EB // 128):
            pltpu.sync_copy(phi_v.at[pl.ds(j * 128, 128), :],
                            acc.at[idx_v.at[j]], add=True)
        return carry

    lax.fori_loop(0, BLOCKS_PER_TILE, body, 0)
    plsc.subcore_barrier()
    pltpu.sync_copy(acc.at[pl.ds(rbase, ROWS_PER_TILE), :],
                    out_hbm.at[pl.ds(rbase, ROWS_PER_TILE), pl.ds(col, 128)])


@functools.lru_cache(maxsize=None)
def _sc_scatter_fn():
    mesh = plsc.VectorSubcoreMesh(core_axis_name="c", subcore_axis_name="s")
    return pl.kernel(
        _sc_scatter_body,
        out_type=jax.ShapeDtypeStruct((N_ACC, PHI_PAD), jnp.float32),
        mesh=mesh,
        scratch_types=[
            pltpu.VMEM((EB // 128, 128), jnp.int32),
            pltpu.VMEM((EB, 128), jnp.float32),
            pltpu.VMEM_SHARED((N_ACC, 128), jnp.float32),
        ],
    )


def _sc_scatter(phi, src_pad, zeros):
    return _sc_scatter_fn()(phi, src_pad, zeros)


_SYM_FLAT = [0, 1, 2, 1, 3, 4, 2, 4, 5]


def _proj_body(p_ref, wvt, wddt, a_ref, v_ref, d_ref):
    A = p_ref[...]
    a_ref[...] = A[:, 0:16]
    wv = wvt[...]
    wdd = wddt[...]
    T = [jnp.dot(A[:, 16 * (1 + k):16 * (2 + k)], wv,
                 preferred_element_type=jnp.float32) for k in range(3)]
    ic3 = lax.broadcasted_iota(jnp.int32, (3,), 0)
    v_ref[...] = sum(T[k][:, :, None] * (ic3 == k).astype(jnp.float32)
                     for k in range(3))
    D = [jnp.dot(A[:, 16 * (4 + k):16 * (5 + k)], wdd,
                 preferred_element_type=jnp.float32) for k in range(6)]
    ic9 = lax.broadcasted_iota(jnp.int32, (9,), 0)
    d_ref[...] = sum(D[_SYM_FLAT[j]][:, :, None] * (ic9 == j).astype(jnp.float32)
                     for j in range(9))


def _stage3(parts, wvt, wddt):
    full = lambda s: pl.BlockSpec(s, lambda i: (0,) * len(s))
    return pl.pallas_call(
        _proj_body,
        grid=(N_NODES // BN,),
        in_specs=[
            pl.BlockSpec((BN, PHI_PAD), lambda i: (i, 0)),
            full((16, 16)), full((16, 16)),
        ],
        out_specs=[
            pl.BlockSpec((BN, 16), lambda i: (i, 0)),
            pl.BlockSpec((BN, 16, 3), lambda i: (i, 0, 0)),
            pl.BlockSpec((BN, 16, 9), lambda i: (i, 0, 0)),
        ],
        out_shape=[
            jax.ShapeDtypeStruct((N_NODES, 16), jnp.float32),
            jax.ShapeDtypeStruct((N_NODES, 16, 3), jnp.float32),
            jax.ShapeDtypeStruct((N_NODES, 16, 9), jnp.float32),
        ],
    )(parts, wvt, wddt)


def kernel(r_ij, edge_index, W0, b0, Wd, W1, b1, W2, b2, W3, b3, Wv, Wdd):
    src = edge_index[0].astype(jnp.int32)
    src_pad = jnp.concatenate(
        [src, jnp.full((E_PAD - E,), N_NODES, jnp.int32)]).reshape(E_PAD // 128, 128)
    r_pad = jnp.pad(r_ij, ((0, E_PAD - E), (0, 0)))
    phi = _stage1(r_pad, W0.T, b0.reshape(1, 16), Wd.T,
                  W1.T, b1.reshape(1, 32), W2.T, b2.reshape(1, 32),
                  W3.T, b3.reshape(1, 16))
    acc = _sc_scatter(phi, src_pad, jnp.zeros((N_ACC, 128), jnp.float32))
    a_a, out_v, out_d = _stage3(acc, Wv.T, Wdd.T)
    return a_a, out_v, out_d.reshape(N_NODES, 16, 3, 3)


# speedup vs baseline: 70.2550x; 2.0946x over previous
import functools

import jax
import jax.numpy as jnp
import numpy as np
from jax import lax
from jax.experimental import pallas as pl
from jax.experimental.pallas import tpu as pltpu
from jax.experimental.pallas import tpu_sc as plsc

N_NODES = 10000
E = 160000
R0 = 5.0

NC = 2
NS = 16
EB = 256
BLOCKS_PER_TILE = 40
E_PAD = NS * BLOCKS_PER_TILE * EB
ROWS_PER_TILE = 632
N_ACC = NS * ROWS_PER_TILE
PHI_PAD = 256
BE = 2048
BN = 2000


_U = np.zeros((3, 16), np.float32)
_V = np.zeros((3, 16), np.float32)
_BU = np.zeros((1, 16), np.float32)
_BV = np.zeros((1, 16), np.float32)
_BU[0, 0] = 1.0
_BV[0, 0:4] = 1.0
for _k, _c in [(1, 0), (2, 1), (3, 2), (4, 0), (5, 0), (6, 0), (7, 1), (8, 1), (9, 2)]:
    _U[_c, _k] = 1.0
for _k, _c in [(4, 0), (5, 1), (6, 2), (7, 1), (8, 2), (9, 2)]:
    _V[_c, _k] = 1.0

_R = np.zeros((16, PHI_PAD), np.float32)
_T = np.zeros((16, PHI_PAD), np.float32)
for _a in range(16):
    for _k in range(16):
        _R[_a, _k * 16 + _a] = 1.0
        _T[_k, _k * 16 + _a] = 1.0

_SYM_FLAT = [0, 1, 2, 1, 3, 4, 2, 4, 5]
_SELV = np.zeros((16, 3), np.float32)
for _c in range(3):
    _SELV[1 + _c, _c] = 1.0
_SELD = np.zeros((16, 9), np.float32)
for _rs in range(9):
    _SELD[4 + _SYM_FLAT[_rs], _rs] = 1.0


def _leaky(x):
    return jnp.where(x >= 0, x, 0.1 * x)


def _phi_body(r_ref, w0t, b0, wdt, w1t, b1, w2t, b2, w3t, b3,
              um, bu, vm, bv, rm, tm, phi_ref):
    r = r_ref[...]
    d = jnp.sqrt(jnp.sum(r * r, axis=1, keepdims=True) + 1e-12)
    x = d / R0
    mu = lax.broadcasted_iota(jnp.int32, (1, 8), 1).astype(jnp.float32) / 7.0
    g = jnp.exp(-0.5 * ((x - mu) * 8.0) ** 2)
    h = jnp.dot(g, w0t[...], preferred_element_type=jnp.float32) + b0[...]
    direct = jnp.dot(h, wdt[...], preferred_element_type=jnp.float32)
    y = _leaky(jnp.dot(h, w1t[...], preferred_element_type=jnp.float32) + b1[...])
    y = _leaky(jnp.dot(y, w2t[...], preferred_element_type=jnp.float32) + b2[...])
    y = jnp.dot(y, w3t[...], preferred_element_type=jnp.float32) + b3[...]
    enc = direct + y
    rs = r * (7.0 / R0)
    n2 = jnp.sum(rs * rs, axis=1, keepdims=True)
    rr = rs / jnp.sqrt(1.0 + n2)
    u = jnp.dot(rr, um[...], preferred_element_type=jnp.float32) + bu[...]
    v = jnp.dot(rr, vm[...], preferred_element_type=jnp.float32) + bv[...]
    coef = u * v
    phi_ref[...] = (jnp.dot(enc, rm[...], preferred_element_type=jnp.float32)
                    * jnp.dot(coef, tm[...], preferred_element_type=jnp.float32))


def _stage1(r_pad, w0t, b0, wdt, w1t, b1, w2t, b2, w3t, b3):
    full = lambda s: pl.BlockSpec(s, lambda i: (0,) * len(s))
    return pl.pallas_call(
        _phi_body,
        grid=(E_PAD // BE,),
        in_specs=[
            pl.BlockSpec((BE, 3), lambda i: (i, 0)),
            full((8, 16)), full((1, 16)), full((16, 16)),
            full((16, 32)), full((1, 32)), full((32, 32)), full((1, 32)),
            full((32, 16)), full((1, 16)),
            full((3, 16)), full((1, 16)), full((3, 16)), full((1, 16)),
            full((16, PHI_PAD)), full((16, PHI_PAD)),
        ],
        out_specs=pl.BlockSpec((BE, PHI_PAD), lambda i: (i, 0)),
        out_shape=jax.ShapeDtypeStruct((E_PAD, PHI_PAD), jnp.float32),
    )(r_pad, w0t, b0, wdt, w1t, b1, w2t, b2, w3t, b3,
      jnp.asarray(_U), jnp.asarray(_BU), jnp.asarray(_V), jnp.asarray(_BV),
      jnp.asarray(_R), jnp.asarray(_T))


def _sc_scatter_body(phi_hbm, src_hbm, zero_hbm, out_hbm, idx_v, phi_v, acc):
    c = lax.axis_index("c")
    s = lax.axis_index("s")
    rbase = s * ROWS_PER_TILE
    col = c * 128
    pltpu.sync_copy(zero_hbm.at[pl.ds(rbase, ROWS_PER_TILE), :],
                    acc.at[pl.ds(rbase, ROWS_PER_TILE), :])
    plsc.subcore_barrier()

    def body(i, carry):
        base = s * (BLOCKS_PER_TILE * EB) + i * EB
        pltpu.sync_copy(
            src_hbm.at[pl.ds(s * (BLOCKS_PER_TILE * EB // 128) + i * (EB // 128),
                             EB // 128), :],
            idx_v)
        pltpu.sync_copy(phi_hbm.at[pl.ds(base, EB), pl.ds(col, 128)], phi_v)
        for j in range(EB // 128):
            pltpu.sync_copy(phi_v.at[pl.ds(j * 128, 128), :],
                            acc.at[idx_v.at[j]], add=True)
        return carry

    lax.fori_loop(0, BLOCKS_PER_TILE, body, 0)
    plsc.subcore_barrier()
    pltpu.sync_copy(acc.at[pl.ds(rbase, ROWS_PER_TILE), :],
                    out_hbm.at[pl.ds(rbase, ROWS_PER_TILE), pl.ds(col, 128)])


@functools.lru_cache(maxsize=None)
def _sc_scatter_fn():
    mesh = plsc.VectorSubcoreMesh(core_axis_name="c", subcore_axis_name="s")
    return pl.kernel(
        _sc_scatter_body,
        out_type=jax.ShapeDtypeStruct((N_ACC, PHI_PAD), jnp.float32),
        mesh=mesh,
        scratch_types=[
            pltpu.VMEM((EB // 128, 128), jnp.int32),
            pltpu.VMEM((EB, 128), jnp.float32),
            pltpu.VMEM_SHARED((N_ACC, 128), jnp.float32),
        ],
    )


def _sc_scatter(phi, src_pad, zeros):
    return _sc_scatter_fn()(phi, src_pad, zeros)


def _proj_body(p_ref, wvb, wdb, a_ref, v_ref, d_ref):
    A = p_ref[...]
    a_ref[...] = A[:, 0:16]
    v_ref[...] = jnp.dot(A, wvb[...], preferred_element_type=jnp.float32)
    d_ref[...] = jnp.dot(A, wdb[...], preferred_element_type=jnp.float32)


def _stage3(acc, wvb, wdb):
    full = lambda s: pl.BlockSpec(s, lambda i: (0,) * len(s))
    return pl.pallas_call(
        _proj_body,
        grid=(N_NODES // BN,),
        in_specs=[
            pl.BlockSpec((BN, PHI_PAD), lambda i: (i, 0)),
            full((PHI_PAD, 48)), full((PHI_PAD, 144)),
        ],
        out_specs=[
            pl.BlockSpec((BN, 16), lambda i: (i, 0)),
            pl.BlockSpec((BN, 48), lambda i: (i, 0)),
            pl.BlockSpec((BN, 144), lambda i: (i, 0)),
        ],
        out_shape=[
            jax.ShapeDtypeStruct((N_NODES, 16), jnp.float32),
            jax.ShapeDtypeStruct((N_NODES, 48), jnp.float32),
            jax.ShapeDtypeStruct((N_NODES, 144), jnp.float32),
        ],
    )(acc, wvb, wdb)


def kernel(r_ij, edge_index, W0, b0, Wd, W1, b1, W2, b2, W3, b3, Wv, Wdd):
    src = edge_index[0].astype(jnp.int32)
    src_pad = jnp.concatenate(
        [src, jnp.full((E_PAD - E,), N_NODES, jnp.int32)]).reshape(E_PAD // 128, 128)
    r_pad = jnp.pad(r_ij, ((0, E_PAD - E), (0, 0)))
    phi = _stage1(r_pad, W0.T, b0.reshape(1, 16), Wd.T,
                  W1.T, b1.reshape(1, 32), W2.T, b2.reshape(1, 32),
                  W3.T, b3.reshape(1, 16))
    acc = _sc_scatter(phi, src_pad, jnp.zeros((N_ACC, 128), jnp.float32))
    wvb = jnp.einsum('kc,va->kavc', jnp.asarray(_SELV), Wv).reshape(PHI_PAD, 48)
    wdb = jnp.einsum('kr,va->kavr', jnp.asarray(_SELD), Wdd).reshape(PHI_PAD, 144)
    a_a, v_flat, d_flat = _stage3(acc, wvb, wdb)
    return (a_a, v_flat.reshape(N_NODES, 16, 3),
            d_flat.reshape(N_NODES, 16, 3, 3))


# speedup vs baseline: 79.9335x; 1.1378x over previous
import functools

import jax
import jax.numpy as jnp
import numpy as np
from jax import lax
from jax.experimental import pallas as pl
from jax.experimental.pallas import tpu as pltpu
from jax.experimental.pallas import tpu_sc as plsc

N_NODES = 10000
E = 160000
R0 = 5.0

NC = 2
NS = 16
EB = 128
BLOCKS_PER_TILE = 80
E_PAD = NS * BLOCKS_PER_TILE * EB
ROWS_PER_TILE = 632
N_ACC = NS * ROWS_PER_TILE
PHI_PAD = 256
BE = 2048
BN = 2000


_U = np.zeros((3, 16), np.float32)
_V = np.zeros((3, 16), np.float32)
_BU = np.zeros((1, 16), np.float32)
_BV = np.zeros((1, 16), np.float32)
_BU[0, 0] = 1.0
_BV[0, 0:4] = 1.0
for _k, _c in [(1, 0), (2, 1), (3, 2), (4, 0), (5, 0), (6, 0), (7, 1), (8, 1), (9, 2)]:
    _U[_c, _k] = 1.0
for _k, _c in [(4, 0), (5, 1), (6, 2), (7, 1), (8, 2), (9, 2)]:
    _V[_c, _k] = 1.0

_R = np.zeros((16, PHI_PAD), np.float32)
_T = np.zeros((16, PHI_PAD), np.float32)
for _a in range(16):
    for _k in range(16):
        _R[_a, _k * 16 + _a] = 1.0
        _T[_k, _k * 16 + _a] = 1.0

_SYM_FLAT = [0, 1, 2, 1, 3, 4, 2, 4, 5]
_SELV = np.zeros((16, 3), np.float32)
for _c in range(3):
    _SELV[1 + _c, _c] = 1.0
_SELD = np.zeros((16, 9), np.float32)
for _rs in range(9):
    _SELD[4 + _SYM_FLAT[_rs], _rs] = 1.0


def _leaky(x):
    return jnp.where(x >= 0, x, 0.1 * x)


def _phi_body(r_ref, w0t, b0, wdt, w1t, b1, w2t, b2, w3t, b3,
              um, bu, vm, bv, rm, tm, phi_ref):
    r = r_ref[...]
    d = jnp.sqrt(jnp.sum(r * r, axis=1, keepdims=True) + 1e-12)
    x = d / R0
    mu = lax.broadcasted_iota(jnp.int32, (1, 8), 1).astype(jnp.float32) / 7.0
    g = jnp.exp(-0.5 * ((x - mu) * 8.0) ** 2)
    h = jnp.dot(g, w0t[...], preferred_element_type=jnp.float32) + b0[...]
    direct = jnp.dot(h, wdt[...], preferred_element_type=jnp.float32)
    y = _leaky(jnp.dot(h, w1t[...], preferred_element_type=jnp.float32) + b1[...])
    y = _leaky(jnp.dot(y, w2t[...], preferred_element_type=jnp.float32) + b2[...])
    y = jnp.dot(y, w3t[...], preferred_element_type=jnp.float32) + b3[...]
    enc = direct + y
    rs = r * (7.0 / R0)
    n2 = jnp.sum(rs * rs, axis=1, keepdims=True)
    rr = rs / jnp.sqrt(1.0 + n2)
    u = jnp.dot(rr, um[...], preferred_element_type=jnp.float32) + bu[...]
    v = jnp.dot(rr, vm[...], preferred_element_type=jnp.float32) + bv[...]
    coef = u * v
    phi_ref[...] = (jnp.dot(enc, rm[...], preferred_element_type=jnp.float32)
                    * jnp.dot(coef, tm[...], preferred_element_type=jnp.float32))


def _stage1(r_pad, w0t, b0, wdt, w1t, b1, w2t, b2, w3t, b3):
    full = lambda s: pl.BlockSpec(s, lambda i: (0,) * len(s))
    return pl.pallas_call(
        _phi_body,
        grid=(E_PAD // BE,),
        in_specs=[
            pl.BlockSpec((BE, 3), lambda i: (i, 0)),
            full((8, 16)), full((1, 16)), full((16, 16)),
            full((16, 32)), full((1, 32)), full((32, 32)), full((1, 32)),
            full((32, 16)), full((1, 16)),
            full((3, 16)), full((1, 16)), full((3, 16)), full((1, 16)),
            full((16, PHI_PAD)), full((16, PHI_PAD)),
        ],
        out_specs=pl.BlockSpec((BE, PHI_PAD), lambda i: (i, 0)),
        out_shape=jax.ShapeDtypeStruct((E_PAD, PHI_PAD), jnp.float32),
    )(r_pad, w0t, b0, wdt, w1t, b1, w2t, b2, w3t, b3,
      jnp.asarray(_U), jnp.asarray(_BU), jnp.asarray(_V), jnp.asarray(_BV),
      jnp.asarray(_R), jnp.asarray(_T))


def _sc_scatter_body(phi_hbm, src_hbm, zero_hbm, out_hbm,
                     idx_v0, idx_v1, phi_v0, phi_v1, acc, sem0, sem1):
    c = lax.axis_index("c")
    s = lax.axis_index("s")
    rbase = s * ROWS_PER_TILE
    col = c * 128
    ib = (idx_v0, idx_v1)
    pb = (phi_v0, phi_v1)
    sems = (sem0, sem1)

    def start_load(i, b):
        pltpu.async_copy(src_hbm.at[pl.ds(s * BLOCKS_PER_TILE + i, 1), :],
                         ib[b], sems[b])
        pltpu.async_copy(phi_hbm.at[pl.ds(s * (BLOCKS_PER_TILE * EB) + i * EB, EB),
                                    pl.ds(col, 128)],
                         pb[b], sems[b])

    def wait_load(b):
        pltpu.make_async_copy(src_hbm.at[pl.ds(0, 1), :], ib[b], sems[b]).wait()
        pltpu.make_async_copy(phi_hbm.at[pl.ds(0, EB), pl.ds(col, 128)],
                              pb[b], sems[b]).wait()

    def scatter(b):
        pltpu.sync_copy(pb[b], acc.at[ib[b].at[0]], add=True)

    start_load(0, 0)
    pltpu.sync_copy(zero_hbm.at[pl.ds(rbase, ROWS_PER_TILE), :],
                    acc.at[pl.ds(rbase, ROWS_PER_TILE), :])
    plsc.subcore_barrier()

    def body(g, carry):
        i0 = 2 * g
        start_load(i0 + 1, 1)
        wait_load(0)
        scatter(0)

        @pl.when(i0 + 2 < BLOCKS_PER_TILE)
        def _():
            start_load(i0 + 2, 0)

        wait_load(1)
        scatter(1)
        return carry

    lax.fori_loop(0, BLOCKS_PER_TILE // 2, body, 0)
    plsc.subcore_barrier()
    pltpu.sync_copy(acc.at[pl.ds(rbase, ROWS_PER_TILE), :],
                    out_hbm.at[pl.ds(rbase, ROWS_PER_TILE), pl.ds(col, 128)])


@functools.lru_cache(maxsize=None)
def _sc_scatter_fn():
    mesh = plsc.VectorSubcoreMesh(core_axis_name="c", subcore_axis_name="s")
    return pl.kernel(
        _sc_scatter_body,
        out_type=jax.ShapeDtypeStruct((N_ACC, PHI_PAD), jnp.float32),
        mesh=mesh,
        scratch_types=[
            pltpu.VMEM((1, 128), jnp.int32),
            pltpu.VMEM((1, 128), jnp.int32),
            pltpu.VMEM((EB, 128), jnp.float32),
            pltpu.VMEM((EB, 128), jnp.float32),
            pltpu.VMEM_SHARED((N_ACC, 128), jnp.float32),
            pltpu.SemaphoreType.DMA,
            pltpu.SemaphoreType.DMA,
        ],
    )


def _sc_scatter(phi, src_pad, zeros):
    return _sc_scatter_fn()(phi, src_pad, zeros)


def _proj_body(p_ref, wvb, wdb, a_ref, v_ref, d_ref):
    A = p_ref[...]
    a_ref[...] = A[:, 0:16]
    v_ref[...] = jnp.dot(A, wvb[...], preferred_element_type=jnp.float32)
    d_ref[...] = jnp.dot(A, wdb[...], preferred_element_type=jnp.float32)


def _stage3(acc, wvb, wdb):
    full = lambda s: pl.BlockSpec(s, lambda i: (0,) * len(s))
    return pl.pallas_call(
        _proj_body,
        grid=(N_NODES // BN,),
        in_specs=[
            pl.BlockSpec((BN, PHI_PAD), lambda i: (i, 0)),
            full((PHI_PAD, 48)), full((PHI_PAD, 144)),
        ],
        out_specs=[
            pl.BlockSpec((BN, 16), lambda i: (i, 0)),
            pl.BlockSpec((BN, 48), lambda i: (i, 0)),
            pl.BlockSpec((BN, 144), lambda i: (i, 0)),
        ],
        out_shape=[
            jax.ShapeDtypeStruct((N_NODES, 16), jnp.float32),
            jax.ShapeDtypeStruct((N_NODES, 48), jnp.float32),
            jax.ShapeDtypeStruct((N_NODES, 144), jnp.float32),
        ],
    )(acc, wvb, wdb)


def kernel(r_ij, edge_index, W0, b0, Wd, W1, b1, W2, b2, W3, b3, Wv, Wdd):
    src = edge_index[0].astype(jnp.int32)
    src_pad = jnp.concatenate(
        [src, jnp.full((E_PAD - E,), N_NODES, jnp.int32)]).reshape(E_PAD // 128, 128)
    r_pad = jnp.pad(r_ij, ((0, E_PAD - E), (0, 0)))
    phi = _stage1(r_pad, W0.T, b0.reshape(1, 16), Wd.T,
                  W1.T, b1.reshape(1, 32), W2.T, b2.reshape(1, 32),
                  W3.T, b3.reshape(1, 16))
    acc = _sc_scatter(phi, src_pad, jnp.zeros((N_ACC, 128), jnp.float32))
    wvb = jnp.einsum('kc,va->kavc', jnp.asarray(_SELV), Wv).reshape(PHI_PAD, 48)
    wdb = jnp.einsum('kr,va->kavr', jnp.asarray(_SELD), Wdd).reshape(PHI_PAD, 144)
    a_a, v_flat, d_flat = _stage3(acc, wvb, wdb)
    return (a_a, v_flat.reshape(N_NODES, 16, 3),
            d_flat.reshape(N_NODES, 16, 3, 3))


# speedup vs baseline: 82.6443x; 1.0339x over previous
import functools

import jax
import jax.numpy as jnp
import numpy as np
from jax import lax
from jax.experimental import pallas as pl
from jax.experimental.pallas import tpu as pltpu
from jax.experimental.pallas import tpu_sc as plsc

N_NODES = 10000
E = 160000
R0 = 5.0

NC = 2
NS = 16
EB = 128
BLOCKS_PER_TILE = 80
E_PAD = NS * BLOCKS_PER_TILE * EB
ROWS_PER_TILE = 632
N_ACC = NS * ROWS_PER_TILE
PHI_PAD = 256
BE = 4096
BN = 2000


_U = np.zeros((3, 16), np.float32)
_V = np.zeros((3, 16), np.float32)
_BU = np.zeros((1, 16), np.float32)
_BV = np.zeros((1, 16), np.float32)
_BU[0, 0] = 1.0
_BV[0, 0:4] = 1.0
for _k, _c in [(1, 0), (2, 1), (3, 2), (4, 0), (5, 0), (6, 0), (7, 1), (8, 1), (9, 2)]:
    _U[_c, _k] = 1.0
for _k, _c in [(4, 0), (5, 1), (6, 2), (7, 1), (8, 2), (9, 2)]:
    _V[_c, _k] = 1.0

_R = np.zeros((16, PHI_PAD), np.float32)
_T = np.zeros((16, PHI_PAD), np.float32)
for _a in range(16):
    for _k in range(16):
        _R[_a, _k * 16 + _a] = 1.0
        _T[_k, _k * 16 + _a] = 1.0

_SYM_FLAT = [0, 1, 2, 1, 3, 4, 2, 4, 5]
_SELV = np.zeros((16, 3), np.float32)
for _c in range(3):
    _SELV[1 + _c, _c] = 1.0
_SELD = np.zeros((16, 9), np.float32)
for _rs in range(9):
    _SELD[4 + _SYM_FLAT[_rs], _rs] = 1.0


_M32 = np.ones((3, 2), np.float32)
_M32[:, 1] = 49.0 / (R0 * R0)
_B2R = np.array([[1e-12, 1.0]], np.float32)
_MUS = np.linspace(0.0, 1.0, 8).astype(np.float32)
_C3 = np.stack([np.full((8,), -32.0 / (R0 * R0), np.float32),
                (64.0 * _MUS / R0).astype(np.float32)]).astype(np.float32)
_C0 = (-32.0 * _MUS * _MUS).reshape(1, 8).astype(np.float32)


def _leaky(x):
    return jnp.where(x >= 0, x, 0.1 * x)


def _dott(x, w):
    return lax.dot_general(x, w, (((1,), (1,)), ((), ())),
                           preferred_element_type=jnp.float32)


def _phi_body(r_ref, w0, b0, wd, w1, b1, w2, b2, w3, b3,
              um, bu, vm, bv, rm, tm, phi_ref):
    r = r_ref[...]
    d = jnp.sqrt(jnp.sum(r * r, axis=1, keepdims=True) + 1e-12)
    x = d / R0
    mu = lax.broadcasted_iota(jnp.int32, (1, 8), 1).astype(jnp.float32) / 7.0
    g = jnp.exp(-0.5 * ((x - mu) * 8.0) ** 2)
    h = _dott(g, w0[...]) + b0[...]
    direct = _dott(h, wd[...])
    y = _leaky(_dott(h, w1[...]) + b1[...])
    y = _leaky(_dott(y, w2[...]) + b2[...])
    y = _dott(y, w3[...]) + b3[...]
    enc = direct + y
    rs = r * (7.0 / R0)
    n2 = jnp.sum(rs * rs, axis=1, keepdims=True)
    rr = rs / jnp.sqrt(1.0 + n2)
    u = jnp.dot(rr, um[...], preferred_element_type=jnp.float32) + bu[...]
    v = jnp.dot(rr, vm[...], preferred_element_type=jnp.float32) + bv[...]
    coef = u * v
    phi_ref[...] = (jnp.dot(enc, rm[...], preferred_element_type=jnp.float32)
                    * jnp.dot(coef, tm[...], preferred_element_type=jnp.float32))


def _stage1(r_pad, w0, b0, wd, w1, b1, w2, b2, w3, b3):
    full = lambda s: pl.BlockSpec(s, lambda i: (0,) * len(s))
    return pl.pallas_call(
        _phi_body,
        grid=(E_PAD // BE,),
        in_specs=[
            pl.BlockSpec((BE, 3), lambda i: (i, 0)),
            full((16, 8)), full((1, 16)), full((16, 16)),
            full((32, 16)), full((1, 32)), full((32, 32)), full((1, 32)),
            full((16, 32)), full((1, 16)),
            full((3, 16)), full((1, 16)), full((3, 16)), full((1, 16)),
            full((16, PHI_PAD)), full((16, PHI_PAD)),
        ],
        out_specs=pl.BlockSpec((BE, PHI_PAD), lambda i: (i, 0)),
        out_shape=jax.ShapeDtypeStruct((E_PAD, PHI_PAD), jnp.float32),
    )(r_pad, w0, b0, wd, w1, b1, w2, b2, w3, b3,
      jnp.asarray(_U), jnp.asarray(_BU), jnp.asarray(_V), jnp.asarray(_BV),
      jnp.asarray(_R), jnp.asarray(_T))


def _sc_scatter_body(phi_hbm, src_hbm, zero_hbm, out_hbm,
                     idx_v0, idx_v1, phi_v0, phi_v1, acc, sem0, sem1):
    c = lax.axis_index("c")
    s = lax.axis_index("s")
    rbase = s * ROWS_PER_TILE
    col = c * 128
    ib = (idx_v0, idx_v1)
    pb = (phi_v0, phi_v1)
    sems = (sem0, sem1)

    def start_load(i, b):
        pltpu.async_copy(src_hbm.at[pl.ds(s * BLOCKS_PER_TILE + i, 1), :],
                         ib[b], sems[b])
        pltpu.async_copy(phi_hbm.at[pl.ds(s * (BLOCKS_PER_TILE * EB) + i * EB, EB),
                                    pl.ds(col, 128)],
                         pb[b], sems[b])

    def wait_load(b):
        pltpu.make_async_copy(src_hbm.at[pl.ds(0, 1), :], ib[b], sems[b]).wait()
        pltpu.make_async_copy(phi_hbm.at[pl.ds(0, EB), pl.ds(col, 128)],
                              pb[b], sems[b]).wait()

    def scatter(b):
        pltpu.sync_copy(pb[b], acc.at[ib[b].at[0]], add=True)

    start_load(0, 0)
    pltpu.sync_copy(zero_hbm.at[pl.ds(rbase, ROWS_PER_TILE), :],
                    acc.at[pl.ds(rbase, ROWS_PER_TILE), :])
    plsc.subcore_barrier()

    def body(g, carry):
        i0 = 2 * g
        start_load(i0 + 1, 1)
        wait_load(0)
        scatter(0)

        @pl.when(i0 + 2 < BLOCKS_PER_TILE)
        def _():
            start_load(i0 + 2, 0)

        wait_load(1)
        scatter(1)
        return carry

    lax.fori_loop(0, BLOCKS_PER_TILE // 2, body, 0)
    plsc.subcore_barrier()
    pltpu.sync_copy(acc.at[pl.ds(rbase, ROWS_PER_TILE), :],
                    out_hbm.at[pl.ds(rbase, ROWS_PER_TILE), pl.ds(col, 128)])


@functools.lru_cache(maxsize=None)
def _sc_scatter_fn():
    mesh = plsc.VectorSubcoreMesh(core_axis_name="c", subcore_axis_name="s")
    return pl.kernel(
        _sc_scatter_body,
        out_type=jax.ShapeDtypeStruct((N_ACC, PHI_PAD), jnp.float32),
        mesh=mesh,
        scratch_types=[
            pltpu.VMEM((1, 128), jnp.int32),
            pltpu.VMEM((1, 128), jnp.int32),
            pltpu.VMEM((EB, 128), jnp.float32),
            pltpu.VMEM((EB, 128), jnp.float32),
            pltpu.VMEM_SHARED((N_ACC, 128), jnp.float32),
            pltpu.SemaphoreType.DMA,
            pltpu.SemaphoreType.DMA,
        ],
    )


def _sc_scatter(phi, src_pad, zeros):
    return _sc_scatter_fn()(phi, src_pad, zeros)


def _proj_body(p_ref, wvb, wdb, a_ref, v_ref, d_ref):
    A = p_ref[...]
    a_ref[...] = A[:, 0:16]
    v_ref[...] = jnp.dot(A, wvb[...], preferred_element_type=jnp.float32)
    d_ref[...] = jnp.dot(A, wdb[...], preferred_element_type=jnp.float32)


def _stage3(acc, wvb, wdb):
    full = lambda s: pl.BlockSpec(s, lambda i: (0,) * len(s))
    return pl.pallas_call(
        _proj_body,
        grid=(N_NODES // BN,),
        in_specs=[
            pl.BlockSpec((BN, PHI_PAD), lambda i: (i, 0)),
            full((PHI_PAD, 48)), full((PHI_PAD, 144)),
        ],
        out_specs=[
            pl.BlockSpec((BN, 16), lambda i: (i, 0)),
            pl.BlockSpec((BN, 48), lambda i: (i, 0)),
            pl.BlockSpec((BN, 144), lambda i: (i, 0)),
        ],
        out_shape=[
            jax.ShapeDtypeStruct((N_NODES, 16), jnp.float32),
            jax.ShapeDtypeStruct((N_NODES, 48), jnp.float32),
            jax.ShapeDtypeStruct((N_NODES, 144), jnp.float32),
        ],
    )(acc, wvb, wdb)


def kernel(r_ij, edge_index, W0, b0, Wd, W1, b1, W2, b2, W3, b3, Wv, Wdd):
    src = edge_index[0].astype(jnp.int32)
    src_pad = jnp.concatenate(
        [src, jnp.full((E_PAD - E,), N_NODES, jnp.int32)]).reshape(E_PAD // 128, 128)
    r_pad = jnp.pad(r_ij, ((0, E_PAD - E), (0, 0)))
    phi = _stage1(r_pad, W0, b0.reshape(1, 16), Wd,
                  W1, b1.reshape(1, 32), W2, b2.reshape(1, 32),
                  W3, b3.reshape(1, 16))
    acc = _sc_scatter(phi, src_pad, jnp.zeros((N_ACC, 128), jnp.float32))
    wvb = jnp.einsum('kc,va->kavc', jnp.asarray(_SELV), Wv).reshape(PHI_PAD, 48)
    wdb = jnp.einsum('kr,va->kavr', jnp.asarray(_SELD), Wdd).reshape(PHI_PAD, 144)
    a_a, v_flat, d_flat = _stage3(acc, wvb, wdb)
    return (a_a, v_flat.reshape(N_NODES, 16, 3),
            d_flat.reshape(N_NODES, 16, 3, 3))
